# Initial kernel scaffold; baseline (speedup 1.0000x reference)
import functools

import jax
import jax.numpy as jnp
from jax import lax
from jax.experimental import pallas as pl
from jax.experimental.pallas import tpu as pltpu
from jax.experimental.pallas import tpu_sc as plsc

NC = 2
NS = 16
NW = NC * NS
LANES = 16
BATCH = 128

_MESH = plsc.VectorSubcoreMesh(core_axis_name="c", subcore_axis_name="s")


def _sc_scatter_add(src_r, dst_r, table, zeros, *, n_rows_out, k_batches):
    @functools.partial(
        pl.kernel,
        out_type=jax.ShapeDtypeStruct((NC, n_rows_out, LANES), jnp.float32),
        mesh=_MESH,
        scratch_types=[
            pltpu.VMEM((k_batches, BATCH), jnp.int32),
            pltpu.VMEM((k_batches, BATCH), jnp.int32),
            pltpu.VMEM((BATCH, LANES), jnp.float32),
            pltpu.VMEM_SHARED((n_rows_out, LANES), jnp.float32),
            pltpu.SemaphoreType.DMA,
        ],
    )
    def k(src_hbm, dst_hbm, t_hbm, z_hbm, out_hbm, src_v, dst_v, rows_v, acc_sh, sem):
        c = lax.axis_index("c")
        s = lax.axis_index("s")
        w = c * NS + s
        rows_per = n_rows_out // NS
        pltpu.sync_copy(z_hbm.at[pl.ds(s * rows_per, rows_per)],
                        acc_sh.at[pl.ds(s * rows_per, rows_per)])
        pltpu.sync_copy(src_hbm.at[w], src_v)
        pltpu.sync_copy(dst_hbm.at[w], dst_v)
        plsc.subcore_barrier()

        def body(j, carry):
            pltpu.async_copy(t_hbm.at[src_v.at[j]], rows_v, sem).wait()
            pltpu.sync_copy(rows_v, acc_sh.at[dst_v.at[j]], add=True)
            return carry

        lax.fori_loop(0, k_batches, body, 0)
        plsc.subcore_barrier()
        pltpu.sync_copy(acc_sh.at[pl.ds(s * rows_per, rows_per)],
                        out_hbm.at[c, pl.ds(s * rows_per, rows_per)])

    return k(src_r, dst_r, table, zeros)


def _sc_degree(dst_r, ones, zeros, *, n_rows_out, k_batches):
    @functools.partial(
        pl.kernel,
        out_type=jax.ShapeDtypeStruct((NC, n_rows_out, LANES), jnp.float32),
        mesh=_MESH,
        scratch_types=[
            pltpu.VMEM((k_batches, BATCH), jnp.int32),
            pltpu.VMEM((BATCH, LANES), jnp.float32),
            pltpu.VMEM_SHARED((n_rows_out, LANES), jnp.float32),
        ],
    )
    def k(dst_hbm, ones_hbm, z_hbm, out_hbm, dst_v, ones_v, acc_sh):
        c = lax.axis_index("c")
        s = lax.axis_index("s")
        w = c * NS + s
        rows_per = n_rows_out // NS
        pltpu.sync_copy(z_hbm.at[pl.ds(s * rows_per, rows_per)],
                        acc_sh.at[pl.ds(s * rows_per, rows_per)])
        pltpu.sync_copy(dst_hbm.at[w], dst_v)
        pltpu.sync_copy(ones_hbm, ones_v)
        plsc.subcore_barrier()

        def body(j, carry):
            pltpu.sync_copy(ones_v, acc_sh.at[dst_v.at[j]], add=True)
            return carry

        lax.fori_loop(0, k_batches, body, 0)
        plsc.subcore_barrier()
        pltpu.sync_copy(acc_sh.at[pl.ds(s * rows_per, rows_per)],
                        out_hbm.at[c, pl.ds(s * rows_per, rows_per)])

    return k(dst_r, ones, zeros)


def _tc_proj1(x, W1, degp, *, n, blk):
    def body(x_ref, w_ref, deg_ref, t1_ref, dinv_ref):
        deg = deg_ref[0] + deg_ref[1] + 1.0
        dinv = lax.rsqrt(deg)
        dinv_ref[...] = dinv
        h = jnp.dot(x_ref[...], w_ref[...], preferred_element_type=jnp.float32)
        t1_ref[...] = h * dinv

    grid = (n // blk,)
    return pl.pallas_call(
        body,
        grid=grid,
        in_specs=[
            pl.BlockSpec((blk, x.shape[1]), lambda i: (i, 0)),
            pl.BlockSpec(W1.shape, lambda i: (0, 0)),
            pl.BlockSpec((NC, blk, LANES), lambda i: (0, i, 0)),
        ],
        out_specs=[
            pl.BlockSpec((blk, LANES), lambda i: (i, 0)),
            pl.BlockSpec((blk, LANES), lambda i: (i, 0)),
        ],
        out_shape=[
            jax.ShapeDtypeStruct((n, LANES), jnp.float32),
            jax.ShapeDtypeStruct((n, LANES), jnp.float32),
        ],
    )(x, W1, degp)


def _tc_mid(p1, t1, dinv, b1, *, n, blk):
    def body(p_ref, t1_ref, dinv_ref, b_ref, t2_ref):
        dinv = dinv_ref[...]
        s = (p_ref[0] + p_ref[1] + t1_ref[...]) * dinv
        h1 = jnp.maximum(s + b_ref[...], 0.0)
        t2_ref[...] = h1 * dinv

    grid = (n // blk,)
    return pl.pallas_call(
        body,
        grid=grid,
        in_specs=[
            pl.BlockSpec((NC, blk, LANES), lambda i: (0, i, 0)),
            pl.BlockSpec((blk, LANES), lambda i: (i, 0)),
            pl.BlockSpec((blk, LANES), lambda i: (i, 0)),
            pl.BlockSpec((1, LANES), lambda i: (0, 0)),
        ],
        out_specs=pl.BlockSpec((blk, LANES), lambda i: (i, 0)),
        out_shape=jax.ShapeDtypeStruct((n, LANES), jnp.float32),
    )(p1, t1, dinv, b1)


def _tc_final(p2, t2, dinv, W2, b2, *, n, blk, out_ch):
    def body(p_ref, t2_ref, dinv_ref, w_ref, b_ref, o_ref):
        g = (p_ref[0] + p_ref[1] + t2_ref[...]) * dinv_ref[...]
        z = jnp.dot(g, w_ref[...], preferred_element_type=jnp.float32) + b_ref[...]
        m = jnp.max(z, axis=1, keepdims=True)
        zs = z - m
        lse = jnp.log(jnp.sum(jnp.exp(zs), axis=1, keepdims=True))
        o_ref[...] = zs - lse

    grid = (n // blk,)
    return pl.pallas_call(
        body,
        grid=grid,
        in_specs=[
            pl.BlockSpec((NC, blk, LANES), lambda i: (0, i, 0)),
            pl.BlockSpec((blk, LANES), lambda i: (i, 0)),
            pl.BlockSpec((blk, LANES), lambda i: (i, 0)),
            pl.BlockSpec(W2.shape, lambda i: (0, 0)),
            pl.BlockSpec((1, out_ch), lambda i: (0, 0)),
        ],
        out_specs=pl.BlockSpec((blk, out_ch), lambda i: (i, 0)),
        out_shape=jax.ShapeDtypeStruct((n, out_ch), jnp.float32),
    )(p2, t2, dinv, W2, b2)


def kernel(x, edge_index, W1, b1, W2, b2):
    n = x.shape[0]
    e = edge_index.shape[1]
    out_ch = W2.shape[1]
    blk = 1000

    n_rows = (n + NS + NS - 1) // NS * NS
    dummy = n

    k_batches = -(-e // (NW * BATCH))
    e_pad = k_batches * NW * BATCH
    src = edge_index[0].astype(jnp.int32)
    dst = edge_index[1].astype(jnp.int32)
    src_r = jnp.concatenate(
        [src, jnp.zeros((e_pad - e,), jnp.int32)]).reshape(NW, k_batches, BATCH)
    dst_r = jnp.concatenate(
        [dst, jnp.full((e_pad - e,), dummy, jnp.int32)]).reshape(NW, k_batches, BATCH)

    zeros = jnp.zeros((n_rows, LANES), jnp.float32)
    ones = jnp.ones((BATCH, LANES), jnp.float32)

    degp = _sc_degree(dst_r, ones, zeros, n_rows_out=n_rows, k_batches=k_batches)

    t1, dinv = _tc_proj1(x, W1, degp[:, :n], n=n, blk=blk)
    p1 = _sc_scatter_add(src_r, dst_r, t1, zeros,
                         n_rows_out=n_rows, k_batches=k_batches)
    t2 = _tc_mid(p1[:, :n], t1, dinv, b1.reshape(1, LANES), n=n, blk=blk)
    p2 = _sc_scatter_add(src_r, dst_r, t2, zeros,
                         n_rows_out=n_rows, k_batches=k_batches)
    out = _tc_final(p2[:, :n], t2, dinv, W2, b2.reshape(1, out_ch),
                    n=n, blk=blk, out_ch=out_ch)
    return out


# speedup vs baseline: 32.5376x; 32.5376x over previous
import functools

import jax
import jax.numpy as jnp
from jax import lax
from jax.experimental import pallas as pl
from jax.experimental.pallas import tpu as pltpu
from jax.experimental.pallas import tpu_sc as plsc

NC = 2
NS = 16
NW = NC * NS
LANES = 16
BATCH = 128

_MESH = plsc.VectorSubcoreMesh(core_axis_name="c", subcore_axis_name="s")


def _sc_scatter_add(src_r, dst_r, table, zeros, *, n_rows_out, k_batches):
    @functools.partial(
        pl.kernel,
        out_type=jax.ShapeDtypeStruct((NC, n_rows_out, LANES), jnp.float32),
        mesh=_MESH,
        scratch_types=[
            pltpu.VMEM((k_batches, BATCH), jnp.int32),
            pltpu.VMEM((k_batches, BATCH), jnp.int32),
            pltpu.VMEM((BATCH, LANES), jnp.float32),
            pltpu.VMEM_SHARED((n_rows_out, LANES), jnp.float32),
            pltpu.SemaphoreType.DMA,
        ],
        compiler_params=pltpu.CompilerParams(use_tc_tiling_on_sc=False),
    )
    def k(src_hbm, dst_hbm, t_hbm, z_hbm, out_hbm, src_v, dst_v, rows_v, acc_sh, sem):
        c = lax.axis_index("c")
        s = lax.axis_index("s")
        w = c * NS + s
        rows_per = n_rows_out // NS
        pltpu.sync_copy(z_hbm.at[pl.ds(s * rows_per, rows_per)],
                        acc_sh.at[pl.ds(s * rows_per, rows_per)])
        pltpu.sync_copy(src_hbm.at[w], src_v)
        pltpu.sync_copy(dst_hbm.at[w], dst_v)
        plsc.subcore_barrier()

        def body(j, carry):
            pltpu.async_copy(t_hbm.at[src_v.at[j]], rows_v, sem).wait()
            pltpu.sync_copy(rows_v, acc_sh.at[dst_v.at[j]], add=True)
            return carry

        lax.fori_loop(0, k_batches, body, 0)
        plsc.subcore_barrier()
        pltpu.sync_copy(acc_sh.at[pl.ds(s * rows_per, rows_per)],
                        out_hbm.at[c, pl.ds(s * rows_per, rows_per)])

    return k(src_r, dst_r, table, zeros)


def _sc_degree(dst_r, ones, zeros, *, n_rows_out, k_batches):
    @functools.partial(
        pl.kernel,
        out_type=jax.ShapeDtypeStruct((NC, n_rows_out, LANES), jnp.float32),
        mesh=_MESH,
        scratch_types=[
            pltpu.VMEM((k_batches, BATCH), jnp.int32),
            pltpu.VMEM((BATCH, LANES), jnp.float32),
            pltpu.VMEM_SHARED((n_rows_out, LANES), jnp.float32),
        ],
        compiler_params=pltpu.CompilerParams(use_tc_tiling_on_sc=False),
    )
    def k(dst_hbm, ones_hbm, z_hbm, out_hbm, dst_v, ones_v, acc_sh):
        c = lax.axis_index("c")
        s = lax.axis_index("s")
        w = c * NS + s
        rows_per = n_rows_out // NS
        pltpu.sync_copy(z_hbm.at[pl.ds(s * rows_per, rows_per)],
                        acc_sh.at[pl.ds(s * rows_per, rows_per)])
        pltpu.sync_copy(dst_hbm.at[w], dst_v)
        pltpu.sync_copy(ones_hbm, ones_v)
        plsc.subcore_barrier()

        def body(j, carry):
            pltpu.sync_copy(ones_v, acc_sh.at[dst_v.at[j]], add=True)
            return carry

        lax.fori_loop(0, k_batches, body, 0)
        plsc.subcore_barrier()
        pltpu.sync_copy(acc_sh.at[pl.ds(s * rows_per, rows_per)],
                        out_hbm.at[c, pl.ds(s * rows_per, rows_per)])

    return k(dst_r, ones, zeros)


def _tc_proj1(x, W1, degp, *, n, blk):
    def body(x_ref, w_ref, deg_ref, t1_ref, dinv_ref):
        deg = deg_ref[0] + deg_ref[1] + 1.0
        dinv = lax.rsqrt(deg)
        dinv_ref[...] = dinv
        h = jnp.dot(x_ref[...], w_ref[...], preferred_element_type=jnp.float32)
        t1_ref[...] = h * dinv

    grid = (n // blk,)
    return pl.pallas_call(
        body,
        grid=grid,
        in_specs=[
            pl.BlockSpec((blk, x.shape[1]), lambda i: (i, 0)),
            pl.BlockSpec(W1.shape, lambda i: (0, 0)),
            pl.BlockSpec((NC, blk, LANES), lambda i: (0, i, 0)),
        ],
        out_specs=[
            pl.BlockSpec((blk, LANES), lambda i: (i, 0)),
            pl.BlockSpec((blk, LANES), lambda i: (i, 0)),
        ],
        out_shape=[
            jax.ShapeDtypeStruct((n, LANES), jnp.float32),
            jax.ShapeDtypeStruct((n, LANES), jnp.float32),
        ],
    )(x, W1, degp)


def _tc_mid(p1, t1, dinv, b1, *, n, blk):
    def body(p_ref, t1_ref, dinv_ref, b_ref, t2_ref):
        dinv = dinv_ref[...]
        s = (p_ref[0] + p_ref[1] + t1_ref[...]) * dinv
        h1 = jnp.maximum(s + b_ref[...], 0.0)
        t2_ref[...] = h1 * dinv

    grid = (n // blk,)
    return pl.pallas_call(
        body,
        grid=grid,
        in_specs=[
            pl.BlockSpec((NC, blk, LANES), lambda i: (0, i, 0)),
            pl.BlockSpec((blk, LANES), lambda i: (i, 0)),
            pl.BlockSpec((blk, LANES), lambda i: (i, 0)),
            pl.BlockSpec((1, LANES), lambda i: (0, 0)),
        ],
        out_specs=pl.BlockSpec((blk, LANES), lambda i: (i, 0)),
        out_shape=jax.ShapeDtypeStruct((n, LANES), jnp.float32),
    )(p1, t1, dinv, b1)


def _tc_final(p2, t2, dinv, W2, b2, *, n, blk, out_ch):
    def body(p_ref, t2_ref, dinv_ref, w_ref, b_ref, o_ref):
        g = (p_ref[0] + p_ref[1] + t2_ref[...]) * dinv_ref[...]
        z = jnp.dot(g, w_ref[...], preferred_element_type=jnp.float32) + b_ref[...]
        m = jnp.max(z, axis=1, keepdims=True)
        zs = z - m
        lse = jnp.log(jnp.sum(jnp.exp(zs), axis=1, keepdims=True))
        o_ref[...] = zs - lse

    grid = (n // blk,)
    return pl.pallas_call(
        body,
        grid=grid,
        in_specs=[
            pl.BlockSpec((NC, blk, LANES), lambda i: (0, i, 0)),
            pl.BlockSpec((blk, LANES), lambda i: (i, 0)),
            pl.BlockSpec((blk, LANES), lambda i: (i, 0)),
            pl.BlockSpec(W2.shape, lambda i: (0, 0)),
            pl.BlockSpec((1, out_ch), lambda i: (0, 0)),
        ],
        out_specs=pl.BlockSpec((blk, out_ch), lambda i: (i, 0)),
        out_shape=jax.ShapeDtypeStruct((n, out_ch), jnp.float32),
    )(p2, t2, dinv, W2, b2)


def kernel(x, edge_index, W1, b1, W2, b2):
    n = x.shape[0]
    e = edge_index.shape[1]
    out_ch = W2.shape[1]
    blk = 1000

    n_rows = (n + 1 + NS * 8 - 1) // (NS * 8) * (NS * 8)
    dummy = n

    k_batches = -(-e // (NW * BATCH))
    e_pad = k_batches * NW * BATCH
    src = edge_index[0].astype(jnp.int32)
    dst = edge_index[1].astype(jnp.int32)
    src_r = jnp.concatenate(
        [src, jnp.zeros((e_pad - e,), jnp.int32)]).reshape(NW, k_batches, BATCH)
    dst_r = jnp.concatenate(
        [dst, jnp.full((e_pad - e,), dummy, jnp.int32)]).reshape(NW, k_batches, BATCH)

    zeros = jnp.zeros((n_rows, LANES), jnp.float32)
    ones = jnp.ones((BATCH, LANES), jnp.float32)

    degp = _sc_degree(dst_r, ones, zeros, n_rows_out=n_rows, k_batches=k_batches)

    t1, dinv = _tc_proj1(x, W1, degp[:, :n], n=n, blk=blk)
    p1 = _sc_scatter_add(src_r, dst_r, t1, zeros,
                         n_rows_out=n_rows, k_batches=k_batches)
    t2 = _tc_mid(p1[:, :n], t1, dinv, b1.reshape(1, LANES), n=n, blk=blk)
    p2 = _sc_scatter_add(src_r, dst_r, t2, zeros,
                         n_rows_out=n_rows, k_batches=k_batches)
    out = _tc_final(p2[:, :n], t2, dinv, W2, b2.reshape(1, out_ch),
                    n=n, blk=blk, out_ch=out_ch)
    return out


# speedup vs baseline: 36.6446x; 1.1262x over previous
import functools

import jax
import jax.numpy as jnp
from jax import lax
from jax.experimental import pallas as pl
from jax.experimental.pallas import tpu as pltpu
from jax.experimental.pallas import tpu_sc as plsc

NC = 2
NS = 16
NW = NC * NS
LANES = 16
BATCH = 128

_MESH = plsc.VectorSubcoreMesh(core_axis_name="c", subcore_axis_name="s")


def _sc_scatter_add(src_r, dst_r, table, zeros, *, n_rows_out, k_batches):
    @functools.partial(
        pl.kernel,
        out_type=jax.ShapeDtypeStruct((NC, n_rows_out, LANES), jnp.float32),
        mesh=_MESH,
        scratch_types=[
            pltpu.VMEM((k_batches, BATCH), jnp.int32),
            pltpu.VMEM((k_batches, BATCH), jnp.int32),
            pltpu.VMEM((2, BATCH, LANES), jnp.float32),
            pltpu.VMEM_SHARED((n_rows_out, LANES), jnp.float32),
            pltpu.SemaphoreType.DMA((2,)),
        ],
        compiler_params=pltpu.CompilerParams(use_tc_tiling_on_sc=False),
    )
    def k(src_hbm, dst_hbm, t_hbm, z_hbm, out_hbm, src_v, dst_v, rows_v, acc_sh, gsem):
        c = lax.axis_index("c")
        s = lax.axis_index("s")
        w = c * NS + s
        rows_per = n_rows_out // NS
        pltpu.sync_copy(z_hbm.at[pl.ds(s * rows_per, rows_per)],
                        acc_sh.at[pl.ds(s * rows_per, rows_per)])
        pltpu.sync_copy(src_hbm.at[w], src_v)
        pltpu.sync_copy(dst_hbm.at[w], dst_v)
        plsc.subcore_barrier()

        pltpu.async_copy(t_hbm.at[src_v.at[0]], rows_v.at[0], gsem.at[0])
        pltpu.async_copy(t_hbm.at[src_v.at[1]], rows_v.at[1], gsem.at[1])

        def body(step, carry):
            for b in range(2):
                j = step * 2 + b
                pltpu.make_async_copy(
                    t_hbm.at[src_v.at[j]], rows_v.at[b], gsem.at[b]).wait()
                pltpu.sync_copy(rows_v.at[b], acc_sh.at[dst_v.at[j]], add=True)

                def prefetch(jj=j, bb=b):
                    pltpu.async_copy(
                        t_hbm.at[src_v.at[jj + 2]], rows_v.at[bb], gsem.at[bb])

                pl.when(j + 2 < k_batches)(prefetch)
            return carry

        lax.fori_loop(0, k_batches // 2, body, 0)
        plsc.subcore_barrier()
        pltpu.sync_copy(acc_sh.at[pl.ds(s * rows_per, rows_per)],
                        out_hbm.at[c, pl.ds(s * rows_per, rows_per)])

    return k(src_r, dst_r, table, zeros)


def _sc_degree(dst_r, ones, zeros, *, n_rows_out, k_batches):
    @functools.partial(
        pl.kernel,
        out_type=jax.ShapeDtypeStruct((NC, n_rows_out, LANES), jnp.float32),
        mesh=_MESH,
        scratch_types=[
            pltpu.VMEM((k_batches, BATCH), jnp.int32),
            pltpu.VMEM((BATCH, LANES), jnp.float32),
            pltpu.VMEM_SHARED((n_rows_out, LANES), jnp.float32),
            pltpu.SemaphoreType.DMA,
        ],
        compiler_params=pltpu.CompilerParams(use_tc_tiling_on_sc=False),
    )
    def k(dst_hbm, ones_hbm, z_hbm, out_hbm, dst_v, ones_v, acc_sh, ssem):
        c = lax.axis_index("c")
        s = lax.axis_index("s")
        w = c * NS + s
        rows_per = n_rows_out // NS
        pltpu.sync_copy(z_hbm.at[pl.ds(s * rows_per, rows_per)],
                        acc_sh.at[pl.ds(s * rows_per, rows_per)])
        pltpu.sync_copy(dst_hbm.at[w], dst_v)
        pltpu.sync_copy(ones_hbm, ones_v)
        plsc.subcore_barrier()

        def body(step, carry):
            for b in range(8):
                pltpu.async_copy(ones_v, acc_sh.at[dst_v.at[step * 8 + b]],
                                 ssem, add=True)
            for b in range(8):
                pltpu.make_async_copy(
                    ones_v, acc_sh.at[dst_v.at[0]], ssem).wait()
            return carry

        lax.fori_loop(0, k_batches // 8, body, 0)
        plsc.subcore_barrier()
        pltpu.sync_copy(acc_sh.at[pl.ds(s * rows_per, rows_per)],
                        out_hbm.at[c, pl.ds(s * rows_per, rows_per)])

    return k(dst_r, ones, zeros)


def _tc_proj1(x, W1, degp, *, n, blk):
    def body(x_ref, w_ref, deg_ref, t1_ref, dinv_ref):
        deg = deg_ref[0] + deg_ref[1] + 1.0
        dinv = lax.rsqrt(deg)
        dinv_ref[...] = dinv
        h = jnp.dot(x_ref[...], w_ref[...], preferred_element_type=jnp.float32)
        t1_ref[...] = h * dinv

    grid = (n // blk,)
    return pl.pallas_call(
        body,
        grid=grid,
        in_specs=[
            pl.BlockSpec((blk, x.shape[1]), lambda i: (i, 0)),
            pl.BlockSpec(W1.shape, lambda i: (0, 0)),
            pl.BlockSpec((NC, blk, LANES), lambda i: (0, i, 0)),
        ],
        out_specs=[
            pl.BlockSpec((blk, LANES), lambda i: (i, 0)),
            pl.BlockSpec((blk, LANES), lambda i: (i, 0)),
        ],
        out_shape=[
            jax.ShapeDtypeStruct((n, LANES), jnp.float32),
            jax.ShapeDtypeStruct((n, LANES), jnp.float32),
        ],
    )(x, W1, degp)


def _tc_mid(p1, t1, dinv, b1, *, n, blk):
    def body(p_ref, t1_ref, dinv_ref, b_ref, t2_ref):
        dinv = dinv_ref[...]
        s = (p_ref[0] + p_ref[1] + t1_ref[...]) * dinv
        h1 = jnp.maximum(s + b_ref[...], 0.0)
        t2_ref[...] = h1 * dinv

    grid = (n // blk,)
    return pl.pallas_call(
        body,
        grid=grid,
        in_specs=[
            pl.BlockSpec((NC, blk, LANES), lambda i: (0, i, 0)),
            pl.BlockSpec((blk, LANES), lambda i: (i, 0)),
            pl.BlockSpec((blk, LANES), lambda i: (i, 0)),
            pl.BlockSpec((1, LANES), lambda i: (0, 0)),
        ],
        out_specs=pl.BlockSpec((blk, LANES), lambda i: (i, 0)),
        out_shape=jax.ShapeDtypeStruct((n, LANES), jnp.float32),
    )(p1, t1, dinv, b1)


def _tc_final(p2, t2, dinv, W2, b2, *, n, blk, out_ch):
    def body(p_ref, t2_ref, dinv_ref, w_ref, b_ref, o_ref):
        g = (p_ref[0] + p_ref[1] + t2_ref[...]) * dinv_ref[...]
        z = jnp.dot(g, w_ref[...], preferred_element_type=jnp.float32) + b_ref[...]
        m = jnp.max(z, axis=1, keepdims=True)
        zs = z - m
        lse = jnp.log(jnp.sum(jnp.exp(zs), axis=1, keepdims=True))
        o_ref[...] = zs - lse

    grid = (n // blk,)
    return pl.pallas_call(
        body,
        grid=grid,
        in_specs=[
            pl.BlockSpec((NC, blk, LANES), lambda i: (0, i, 0)),
            pl.BlockSpec((blk, LANES), lambda i: (i, 0)),
            pl.BlockSpec((blk, LANES), lambda i: (i, 0)),
            pl.BlockSpec(W2.shape, lambda i: (0, 0)),
            pl.BlockSpec((1, out_ch), lambda i: (0, 0)),
        ],
        out_specs=pl.BlockSpec((blk, out_ch), lambda i: (i, 0)),
        out_shape=jax.ShapeDtypeStruct((n, out_ch), jnp.float32),
    )(p2, t2, dinv, W2, b2)


def kernel(x, edge_index, W1, b1, W2, b2):
    n = x.shape[0]
    e = edge_index.shape[1]
    out_ch = W2.shape[1]
    blk = 1000

    n_rows = (n + 1 + NS * 8 - 1) // (NS * 8) * (NS * 8)
    dummy = n

    k_batches = -(-e // (NW * BATCH * 8)) * 8
    e_pad = k_batches * NW * BATCH
    src = edge_index[0].astype(jnp.int32)
    dst = edge_index[1].astype(jnp.int32)
    src_r = jnp.concatenate(
        [src, jnp.zeros((e_pad - e,), jnp.int32)]).reshape(NW, k_batches, BATCH)
    dst_r = jnp.concatenate(
        [dst, jnp.full((e_pad - e,), dummy, jnp.int32)]).reshape(NW, k_batches, BATCH)

    zeros = jnp.zeros((n_rows, LANES), jnp.float32)
    ones = jnp.ones((BATCH, LANES), jnp.float32)

    degp = _sc_degree(dst_r, ones, zeros, n_rows_out=n_rows, k_batches=k_batches)

    t1, dinv = _tc_proj1(x, W1, degp[:, :n], n=n, blk=blk)
    p1 = _sc_scatter_add(src_r, dst_r, t1, zeros,
                         n_rows_out=n_rows, k_batches=k_batches)
    t2 = _tc_mid(p1[:, :n], t1, dinv, b1.reshape(1, LANES), n=n, blk=blk)
    p2 = _sc_scatter_add(src_r, dst_r, t2, zeros,
                         n_rows_out=n_rows, k_batches=k_batches)
    out = _tc_final(p2[:, :n], t2, dinv, W2, b2.reshape(1, out_ch),
                    n=n, blk=blk, out_ch=out_ch)
    return out


# speedup vs baseline: 42.5600x; 1.1614x over previous
import functools

import jax
import jax.numpy as jnp
from jax import lax
from jax.experimental import pallas as pl
from jax.experimental.pallas import tpu as pltpu
from jax.experimental.pallas import tpu_sc as plsc

NC = 2
NS = 16
NW = NC * NS
LANES = 16
BATCH = 512

_MESH = plsc.VectorSubcoreMesh(core_axis_name="c", subcore_axis_name="s")


def _sc_scatter_add(src_r, dst_r, table, zeros, *, n_rows_out, k_batches):
    @functools.partial(
        pl.kernel,
        out_type=jax.ShapeDtypeStruct((NC, n_rows_out, LANES), jnp.float32),
        mesh=_MESH,
        scratch_types=[
            pltpu.VMEM((k_batches, BATCH), jnp.int32),
            pltpu.VMEM((k_batches, BATCH), jnp.int32),
            pltpu.VMEM((2, BATCH, LANES), jnp.float32),
            pltpu.VMEM_SHARED((n_rows_out, LANES), jnp.float32),
            pltpu.SemaphoreType.DMA((2,)),
        ],
        compiler_params=pltpu.CompilerParams(use_tc_tiling_on_sc=False),
    )
    def k(src_hbm, dst_hbm, t_hbm, z_hbm, out_hbm, src_v, dst_v, rows_v, acc_sh, gsem):
        c = lax.axis_index("c")
        s = lax.axis_index("s")
        w = c * NS + s
        rows_per = n_rows_out // NS
        pltpu.sync_copy(z_hbm.at[pl.ds(s * rows_per, rows_per)],
                        acc_sh.at[pl.ds(s * rows_per, rows_per)])
        pltpu.sync_copy(src_hbm.at[w], src_v)
        pltpu.sync_copy(dst_hbm.at[w], dst_v)
        plsc.subcore_barrier()

        pltpu.async_copy(t_hbm.at[src_v.at[0]], rows_v.at[0], gsem.at[0])
        pltpu.async_copy(t_hbm.at[src_v.at[1]], rows_v.at[1], gsem.at[1])

        def body(step, carry):
            for b in range(2):
                j = step * 2 + b
                pltpu.make_async_copy(
                    t_hbm.at[src_v.at[j]], rows_v.at[b], gsem.at[b]).wait()
                pltpu.sync_copy(rows_v.at[b], acc_sh.at[dst_v.at[j]], add=True)

                def prefetch(jj=j, bb=b):
                    pltpu.async_copy(
                        t_hbm.at[src_v.at[jj + 2]], rows_v.at[bb], gsem.at[bb])

                pl.when(j + 2 < k_batches)(prefetch)
            return carry

        lax.fori_loop(0, k_batches // 2, body, 0)
        plsc.subcore_barrier()
        pltpu.sync_copy(acc_sh.at[pl.ds(s * rows_per, rows_per)],
                        out_hbm.at[c, pl.ds(s * rows_per, rows_per)])

    return k(src_r, dst_r, table, zeros)


def _sc_degree(dst_r, ones, zeros, *, n_rows_out, k_batches):
    @functools.partial(
        pl.kernel,
        out_type=jax.ShapeDtypeStruct((NC, n_rows_out, LANES), jnp.float32),
        mesh=_MESH,
        scratch_types=[
            pltpu.VMEM((k_batches, BATCH), jnp.int32),
            pltpu.VMEM((BATCH, LANES), jnp.float32),
            pltpu.VMEM_SHARED((n_rows_out, LANES), jnp.float32),
            pltpu.SemaphoreType.DMA,
        ],
        compiler_params=pltpu.CompilerParams(use_tc_tiling_on_sc=False),
    )
    def k(dst_hbm, ones_hbm, z_hbm, out_hbm, dst_v, ones_v, acc_sh, ssem):
        c = lax.axis_index("c")
        s = lax.axis_index("s")
        w = c * NS + s
        rows_per = n_rows_out // NS
        pltpu.sync_copy(z_hbm.at[pl.ds(s * rows_per, rows_per)],
                        acc_sh.at[pl.ds(s * rows_per, rows_per)])
        pltpu.sync_copy(dst_hbm.at[w], dst_v)
        pltpu.sync_copy(ones_hbm, ones_v)
        plsc.subcore_barrier()

        def body(step, carry):
            for b in range(4):
                pltpu.async_copy(ones_v, acc_sh.at[dst_v.at[step * 4 + b]],
                                 ssem, add=True)
            for b in range(4):
                pltpu.make_async_copy(
                    ones_v, acc_sh.at[dst_v.at[0]], ssem).wait()
            return carry

        lax.fori_loop(0, k_batches // 4, body, 0)
        plsc.subcore_barrier()
        pltpu.sync_copy(acc_sh.at[pl.ds(s * rows_per, rows_per)],
                        out_hbm.at[c, pl.ds(s * rows_per, rows_per)])

    return k(dst_r, ones, zeros)


def _tc_matmul1(x, W1, *, n, blk):
    def body(x_ref, w_ref, h_ref):
        h_ref[...] = jnp.dot(x_ref[...], w_ref[...],
                             preferred_element_type=jnp.float32)

    grid = (n // blk,)
    return pl.pallas_call(
        body,
        grid=grid,
        in_specs=[
            pl.BlockSpec((blk, x.shape[1]), lambda i: (i, 0)),
            pl.BlockSpec(W1.shape, lambda i: (0, 0)),
        ],
        out_specs=pl.BlockSpec((blk, LANES), lambda i: (i, 0)),
        out_shape=jax.ShapeDtypeStruct((n, LANES), jnp.float32),
    )(x, W1)


def _tc_scale1(h, degp, *, n, blk):
    def body(h_ref, deg_ref, t1_ref, dinv_ref):
        deg = deg_ref[0] + deg_ref[1] + 1.0
        dinv = lax.rsqrt(deg)
        dinv_ref[...] = dinv
        t1_ref[...] = h_ref[...] * dinv

    grid = (n // blk,)
    return pl.pallas_call(
        body,
        grid=grid,
        in_specs=[
            pl.BlockSpec((blk, LANES), lambda i: (i, 0)),
            pl.BlockSpec((NC, blk, LANES), lambda i: (0, i, 0)),
        ],
        out_specs=[
            pl.BlockSpec((blk, LANES), lambda i: (i, 0)),
            pl.BlockSpec((blk, LANES), lambda i: (i, 0)),
        ],
        out_shape=[
            jax.ShapeDtypeStruct((n, LANES), jnp.float32),
            jax.ShapeDtypeStruct((n, LANES), jnp.float32),
        ],
    )(h, degp)


def _tc_mid(p1, t1, dinv, b1, *, n, blk):
    def body(p_ref, t1_ref, dinv_ref, b_ref, t2_ref):
        dinv = dinv_ref[...]
        s = (p_ref[0] + p_ref[1] + t1_ref[...]) * dinv
        h1 = jnp.maximum(s + b_ref[...], 0.0)
        t2_ref[...] = h1 * dinv

    grid = (n // blk,)
    return pl.pallas_call(
        body,
        grid=grid,
        in_specs=[
            pl.BlockSpec((NC, blk, LANES), lambda i: (0, i, 0)),
            pl.BlockSpec((blk, LANES), lambda i: (i, 0)),
            pl.BlockSpec((blk, LANES), lambda i: (i, 0)),
            pl.BlockSpec((1, LANES), lambda i: (0, 0)),
        ],
        out_specs=pl.BlockSpec((blk, LANES), lambda i: (i, 0)),
        out_shape=jax.ShapeDtypeStruct((n, LANES), jnp.float32),
    )(p1, t1, dinv, b1)


def _tc_final(p2, t2, dinv, W2, b2, *, n, blk, out_ch):
    def body(p_ref, t2_ref, dinv_ref, w_ref, b_ref, o_ref):
        g = (p_ref[0] + p_ref[1] + t2_ref[...]) * dinv_ref[...]
        z = jnp.dot(g, w_ref[...], preferred_element_type=jnp.float32) + b_ref[...]
        m = jnp.max(z, axis=1, keepdims=True)
        zs = z - m
        lse = jnp.log(jnp.sum(jnp.exp(zs), axis=1, keepdims=True))
        o_ref[...] = zs - lse

    grid = (n // blk,)
    return pl.pallas_call(
        body,
        grid=grid,
        in_specs=[
            pl.BlockSpec((NC, blk, LANES), lambda i: (0, i, 0)),
            pl.BlockSpec((blk, LANES), lambda i: (i, 0)),
            pl.BlockSpec((blk, LANES), lambda i: (i, 0)),
            pl.BlockSpec(W2.shape, lambda i: (0, 0)),
            pl.BlockSpec((1, out_ch), lambda i: (0, 0)),
        ],
        out_specs=pl.BlockSpec((blk, out_ch), lambda i: (i, 0)),
        out_shape=jax.ShapeDtypeStruct((n, out_ch), jnp.float32),
    )(p2, t2, dinv, W2, b2)


def kernel(x, edge_index, W1, b1, W2, b2):
    n = x.shape[0]
    e = edge_index.shape[1]
    out_ch = W2.shape[1]
    blk = 1000

    n_rows = (n + 1 + NS * 8 - 1) // (NS * 8) * (NS * 8)
    dummy = n

    k_batches = -(-e // (NW * BATCH * 4)) * 4
    e_pad = k_batches * NW * BATCH
    src = edge_index[0].astype(jnp.int32)
    dst = edge_index[1].astype(jnp.int32)
    src_r = jnp.concatenate(
        [src, jnp.zeros((e_pad - e,), jnp.int32)]).reshape(NW, k_batches, BATCH)
    dst_r = jnp.concatenate(
        [dst, jnp.full((e_pad - e,), dummy, jnp.int32)]).reshape(NW, k_batches, BATCH)

    zeros = jnp.zeros((n_rows, LANES), jnp.float32)
    ones = jnp.ones((BATCH, LANES), jnp.float32)

    h = _tc_matmul1(x, W1, n=n, blk=blk)
    degp = _sc_degree(dst_r, ones, zeros, n_rows_out=n_rows, k_batches=k_batches)

    t1, dinv = _tc_scale1(h, degp, n=n, blk=blk)
    p1 = _sc_scatter_add(src_r, dst_r, t1, zeros,
                         n_rows_out=n_rows, k_batches=k_batches)
    t2 = _tc_mid(p1, t1, dinv, b1.reshape(1, LANES), n=n, blk=blk)
    p2 = _sc_scatter_add(src_r, dst_r, t2, zeros,
                         n_rows_out=n_rows, k_batches=k_batches)
    out = _tc_final(p2, t2, dinv, W2, b2.reshape(1, out_ch),
                    n=n, blk=blk, out_ch=out_ch)
    return out


# speedup vs baseline: 42.8077x; 1.0058x over previous
import functools

import jax
import jax.numpy as jnp
from jax import lax
from jax.experimental import pallas as pl
from jax.experimental.pallas import tpu as pltpu
from jax.experimental.pallas import tpu_sc as plsc

NC = 2
NS = 16
NW = NC * NS
LANES = 16
BATCH = 512

_MESH = plsc.VectorSubcoreMesh(core_axis_name="c", subcore_axis_name="s")


def _sc_scatter_add(src_r, dst_r, table, zeros, *, n_rows_out, k_batches):
    @functools.partial(
        pl.kernel,
        out_type=jax.ShapeDtypeStruct((NC, n_rows_out, LANES), jnp.float32),
        mesh=_MESH,
        scratch_types=[
            pltpu.VMEM((k_batches, BATCH), jnp.int32),
            pltpu.VMEM((k_batches, BATCH), jnp.int32),
            pltpu.VMEM((4, BATCH, LANES), jnp.float32),
            pltpu.VMEM_SHARED((n_rows_out, LANES), jnp.float32),
            pltpu.SemaphoreType.DMA((4,)),
            pltpu.SemaphoreType.DMA((4,)),
        ],
        compiler_params=pltpu.CompilerParams(use_tc_tiling_on_sc=False),
    )
    def k(src_hbm, dst_hbm, t_hbm, z_hbm, out_hbm, src_v, dst_v, rows_v, acc_sh,
          gsem, ssem):
        c = lax.axis_index("c")
        s = lax.axis_index("s")
        w = c * NS + s
        rows_per = n_rows_out // NS
        pltpu.sync_copy(z_hbm.at[pl.ds(s * rows_per, rows_per)],
                        acc_sh.at[pl.ds(s * rows_per, rows_per)])
        pltpu.sync_copy(src_hbm.at[w], src_v)
        pltpu.sync_copy(dst_hbm.at[w], dst_v)
        plsc.subcore_barrier()

        pltpu.async_copy(t_hbm.at[src_v.at[0]], rows_v.at[0], gsem.at[0])
        pltpu.async_copy(t_hbm.at[src_v.at[1]], rows_v.at[1], gsem.at[1])

        def body(step, carry):
            for b in range(4):
                j = step * 4 + b
                b2 = (b + 2) % 4
                pltpu.make_async_copy(
                    t_hbm.at[src_v.at[j]], rows_v.at[b], gsem.at[b]).wait()
                pltpu.async_copy(rows_v.at[b], acc_sh.at[dst_v.at[j]],
                                 ssem.at[b], add=True)

                def wait_sc(jj=j, bb=b2):
                    pltpu.make_async_copy(
                        rows_v.at[bb], acc_sh.at[dst_v.at[jj - 2]],
                        ssem.at[bb]).wait()

                pl.when(j >= 2)(wait_sc)

                def prefetch(jj=j, bb=b2):
                    pltpu.async_copy(
                        t_hbm.at[src_v.at[jj + 2]], rows_v.at[bb], gsem.at[bb])

                pl.when(j + 2 < k_batches)(prefetch)
            return carry

        lax.fori_loop(0, k_batches // 4, body, 0)
        for jj in (k_batches - 2, k_batches - 1):
            pltpu.make_async_copy(
                rows_v.at[jj % 4], acc_sh.at[dst_v.at[jj]],
                ssem.at[jj % 4]).wait()
        plsc.subcore_barrier()
        pltpu.sync_copy(acc_sh.at[pl.ds(s * rows_per, rows_per)],
                        out_hbm.at[c, pl.ds(s * rows_per, rows_per)])

    return k(src_r, dst_r, table, zeros)


def _sc_degree(dst_r, ones, zeros, *, n_rows_out, k_batches):
    @functools.partial(
        pl.kernel,
        out_type=jax.ShapeDtypeStruct((NC, n_rows_out, LANES), jnp.float32),
        mesh=_MESH,
        scratch_types=[
            pltpu.VMEM((k_batches, BATCH), jnp.int32),
            pltpu.VMEM((BATCH, LANES), jnp.float32),
            pltpu.VMEM_SHARED((n_rows_out, LANES), jnp.float32),
            pltpu.SemaphoreType.DMA,
        ],
        compiler_params=pltpu.CompilerParams(use_tc_tiling_on_sc=False),
    )
    def k(dst_hbm, ones_hbm, z_hbm, out_hbm, dst_v, ones_v, acc_sh, ssem):
        c = lax.axis_index("c")
        s = lax.axis_index("s")
        w = c * NS + s
        rows_per = n_rows_out // NS
        pltpu.sync_copy(z_hbm.at[pl.ds(s * rows_per, rows_per)],
                        acc_sh.at[pl.ds(s * rows_per, rows_per)])
        pltpu.sync_copy(dst_hbm.at[w], dst_v)
        pltpu.sync_copy(ones_hbm, ones_v)
        plsc.subcore_barrier()

        def body(step, carry):
            for b in range(4):
                pltpu.async_copy(ones_v, acc_sh.at[dst_v.at[step * 4 + b]],
                                 ssem, add=True)
            for b in range(4):
                pltpu.make_async_copy(
                    ones_v, acc_sh.at[dst_v.at[0]], ssem).wait()
            return carry

        lax.fori_loop(0, k_batches // 4, body, 0)
        plsc.subcore_barrier()
        pltpu.sync_copy(acc_sh.at[pl.ds(s * rows_per, rows_per)],
                        out_hbm.at[c, pl.ds(s * rows_per, rows_per)])

    return k(dst_r, ones, zeros)


def _tc_matmul1(x, W1, *, n, blk):
    def body(x_ref, w_ref, h_ref):
        h_ref[...] = jnp.dot(x_ref[...], w_ref[...],
                             preferred_element_type=jnp.float32)

    grid = (n // blk,)
    return pl.pallas_call(
        body,
        grid=grid,
        in_specs=[
            pl.BlockSpec((blk, x.shape[1]), lambda i: (i, 0)),
            pl.BlockSpec(W1.shape, lambda i: (0, 0)),
        ],
        out_specs=pl.BlockSpec((blk, LANES), lambda i: (i, 0)),
        out_shape=jax.ShapeDtypeStruct((n, LANES), jnp.float32),
    )(x, W1)


def _tc_scale1(h, degp, *, n, blk):
    def body(h_ref, deg_ref, t1_ref, dinv_ref):
        deg = deg_ref[0] + deg_ref[1] + 1.0
        dinv = lax.rsqrt(deg)
        dinv_ref[...] = dinv
        t1_ref[...] = h_ref[...] * dinv

    grid = (n // blk,)
    return pl.pallas_call(
        body,
        grid=grid,
        in_specs=[
            pl.BlockSpec((blk, LANES), lambda i: (i, 0)),
            pl.BlockSpec((NC, blk, LANES), lambda i: (0, i, 0)),
        ],
        out_specs=[
            pl.BlockSpec((blk, LANES), lambda i: (i, 0)),
            pl.BlockSpec((blk, LANES), lambda i: (i, 0)),
        ],
        out_shape=[
            jax.ShapeDtypeStruct((n, LANES), jnp.float32),
            jax.ShapeDtypeStruct((n, LANES), jnp.float32),
        ],
    )(h, degp)


def _tc_mid(p1, t1, dinv, b1, *, n, blk):
    def body(p_ref, t1_ref, dinv_ref, b_ref, t2_ref):
        dinv = dinv_ref[...]
        s = (p_ref[0] + p_ref[1] + t1_ref[...]) * dinv
        h1 = jnp.maximum(s + b_ref[...], 0.0)
        t2_ref[...] = h1 * dinv

    grid = (n // blk,)
    return pl.pallas_call(
        body,
        grid=grid,
        in_specs=[
            pl.BlockSpec((NC, blk, LANES), lambda i: (0, i, 0)),
            pl.BlockSpec((blk, LANES), lambda i: (i, 0)),
            pl.BlockSpec((blk, LANES), lambda i: (i, 0)),
            pl.BlockSpec((1, LANES), lambda i: (0, 0)),
        ],
        out_specs=pl.BlockSpec((blk, LANES), lambda i: (i, 0)),
        out_shape=jax.ShapeDtypeStruct((n, LANES), jnp.float32),
    )(p1, t1, dinv, b1)


def _tc_final(p2, t2, dinv, W2, b2, *, n, blk, out_ch):
    def body(p_ref, t2_ref, dinv_ref, w_ref, b_ref, o_ref):
        g = (p_ref[0] + p_ref[1] + t2_ref[...]) * dinv_ref[...]
        z = jnp.dot(g, w_ref[...], preferred_element_type=jnp.float32) + b_ref[...]
        m = jnp.max(z, axis=1, keepdims=True)
        zs = z - m
        lse = jnp.log(jnp.sum(jnp.exp(zs), axis=1, keepdims=True))
        o_ref[...] = zs - lse

    grid = (n // blk,)
    return pl.pallas_call(
        body,
        grid=grid,
        in_specs=[
            pl.BlockSpec((NC, blk, LANES), lambda i: (0, i, 0)),
            pl.BlockSpec((blk, LANES), lambda i: (i, 0)),
            pl.BlockSpec((blk, LANES), lambda i: (i, 0)),
            pl.BlockSpec(W2.shape, lambda i: (0, 0)),
            pl.BlockSpec((1, out_ch), lambda i: (0, 0)),
        ],
        out_specs=pl.BlockSpec((blk, out_ch), lambda i: (i, 0)),
        out_shape=jax.ShapeDtypeStruct((n, out_ch), jnp.float32),
    )(p2, t2, dinv, W2, b2)


def kernel(x, edge_index, W1, b1, W2, b2):
    n = x.shape[0]
    e = edge_index.shape[1]
    out_ch = W2.shape[1]
    blk = 1000

    n_rows = (n + 1 + NS * 8 - 1) // (NS * 8) * (NS * 8)
    dummy = n

    k_batches = -(-e // (NW * BATCH * 4)) * 4
    e_pad = k_batches * NW * BATCH
    src = edge_index[0].astype(jnp.int32)
    dst = edge_index[1].astype(jnp.int32)
    src_r = jnp.concatenate(
        [src, jnp.zeros((e_pad - e,), jnp.int32)]).reshape(NW, k_batches, BATCH)
    dst_r = jnp.concatenate(
        [dst, jnp.full((e_pad - e,), dummy, jnp.int32)]).reshape(NW, k_batches, BATCH)

    zeros = jnp.zeros((n_rows, LANES), jnp.float32)
    ones = jnp.ones((BATCH, LANES), jnp.float32)

    h = _tc_matmul1(x, W1, n=n, blk=blk)
    degp = _sc_degree(dst_r, ones, zeros, n_rows_out=n_rows, k_batches=k_batches)

    t1, dinv = _tc_scale1(h, degp, n=n, blk=blk)
    p1 = _sc_scatter_add(src_r, dst_r, t1, zeros,
                         n_rows_out=n_rows, k_batches=k_batches)
    t2 = _tc_mid(p1, t1, dinv, b1.reshape(1, LANES), n=n, blk=blk)
    p2 = _sc_scatter_add(src_r, dst_r, t2, zeros,
                         n_rows_out=n_rows, k_batches=k_batches)
    out = _tc_final(p2, t2, dinv, W2, b2.reshape(1, out_ch),
                    n=n, blk=blk, out_ch=out_ch)
    return out
